# Initial kernel scaffold; baseline (speedup 1.0000x reference)
import jax
import jax.numpy as jnp
from jax.experimental import pallas as pl


def kernel(x, tables):
    raise NotImplementedError("write your pallas kernel here")


# speedup vs baseline: 19.5137x; 19.5137x over previous
import functools

import numpy as np
import jax
import jax.numpy as jnp
from jax import lax
from jax.experimental import pallas as pl
from jax.experimental.pallas import tpu as pltpu
from jax.experimental.pallas import tpu_sc as plsc

N_LEVELS = 16
F = 2
LOG2_T = 19
T = 2 ** LOG2_T
HASH_MASK = T - 1
BASE_RES = 16.0
FINEST_RES = 512.0
N_PTS = 262144
B_GROWTH = float(np.exp((np.log(FINEST_RES) - np.log(BASE_RES)) / (N_LEVELS - 1)))
_RES = [float(np.floor(np.float32(BASE_RES * (B_GROWTH ** i)))) for i in range(N_LEVELS)]
_GS = [float(np.float32(2.0) / np.float32(r)) for r in _RES]
P1_I = 2654435761 - 2 ** 32
P2_I = 805459861

NC = 2
NS = 16
NW = NC * NS
PW = N_PTS // NW
C = 1024
NCHUNK = PW // C
GROUPS = C // 16


def _sc_body(xt_hbm, tbl_hbm, out_hbm, x_v, w_v, idx_v, rows_v, lv_v, sem):
    c_id = lax.axis_index("c")
    s_id = lax.axis_index("s")
    wid = s_id * NC + c_id
    iota = lax.iota(jnp.int32, 16)

    def chunk_body(ch, carry):
        base = (wid * NCHUNK + ch) * C
        pltpu.sync_copy(xt_hbm.at[:, pl.ds(base, C)], x_v)

        for lvl in range(N_LEVELS):
            gs = jnp.float32(_GS[lvl])
            lvl_off2 = jnp.int32(2 * lvl * T)

            def phase_a(g, _, gs=gs, lvl_off2=lvl_off2):
                p0 = pl.multiple_of(g * 16, 16)
                bl = []
                for d in range(3):
                    xd = x_v[d, pl.ds(p0, 16)]
                    s = (xd - jnp.float32(-1.0)) / gs
                    bli = s.astype(jnp.int32)
                    blf = bli.astype(jnp.float32)
                    vmin = blf * gs + jnp.float32(-1.0)
                    den = (vmin + gs) - vmin
                    w_v[d, pl.ds(p0, 16)] = (xd - vmin) / den
                    bl.append(bli)
                m0 = bl[0]
                m0b = m0 + jnp.int32(1)
                m1 = bl[1] * jnp.int32(P1_I)
                m1b = m1 + jnp.int32(P1_I)
                m2 = bl[2] * jnp.int32(P2_I)
                m2b = m2 + jnp.int32(P2_I)
                e00 = m0 ^ m1
                e01 = m0 ^ m1b
                e10 = m0b ^ m1
                e11 = m0b ^ m1b
                corners = (e00 ^ m2, e00 ^ m2b, e01 ^ m2, e01 ^ m2b,
                           e10 ^ m2, e10 ^ m2b, e11 ^ m2, e11 ^ m2b)
                for c, h in enumerate(corners):
                    f0 = ((h << 1) & jnp.int32(2 * HASH_MASK)) + lvl_off2
                    idx_v[pl.ds(c * C + p0, 16)] = f0
                    idx_v[pl.ds(8 * C + c * C + p0, 16)] = f0 + jnp.int32(1)
                return 0

            lax.fori_loop(0, GROUPS, phase_a, 0)

            pltpu.async_copy(tbl_hbm.at[idx_v], rows_v, sem).wait()

            def phase_b(g, _):
                p0 = pl.multiple_of(g * 16, 16)
                wx = w_v[0, pl.ds(p0, 16)]
                wy = w_v[1, pl.ds(p0, 16)]
                wz = w_v[2, pl.ds(p0, 16)]
                for f in range(F):
                    e = [rows_v[pl.ds(f * 8 * C + c * C + p0, 16)]
                         for c in range(8)]
                    c00 = e[0] + wx * (e[4] - e[0])
                    c01 = e[1] + wx * (e[5] - e[1])
                    c10 = e[2] + wx * (e[6] - e[2])
                    c11 = e[3] + wx * (e[7] - e[3])
                    c0 = c00 + wy * (c10 - c00)
                    c1 = c01 + wy * (c11 - c01)
                    lv_v[f, pl.ds(p0, 16)] = c0 + wz * (c1 - c0)
                return 0

            lax.fori_loop(0, GROUPS, phase_b, 0)

            pltpu.sync_copy(lv_v, out_hbm.at[pl.ds(2 * lvl, 2), pl.ds(base, C)])
        return carry

    lax.fori_loop(0, NCHUNK, chunk_body, 0)


@functools.lru_cache(maxsize=1)
def _make_sc_call():
    mesh = plsc.VectorSubcoreMesh(
        core_axis_name="c", subcore_axis_name="s", num_cores=NC, num_subcores=NS
    )
    return pl.kernel(
        _sc_body,
        out_type=jax.ShapeDtypeStruct((2 * N_LEVELS, N_PTS), jnp.float32),
        mesh=mesh,
        scratch_types=[
            pltpu.VMEM((3, C), jnp.float32),
            pltpu.VMEM((3, C), jnp.float32),
            pltpu.VMEM((F * 8 * C,), jnp.int32),
            pltpu.VMEM((F * 8 * C,), jnp.float32),
            pltpu.VMEM((F, C), jnp.float32),
            pltpu.SemaphoreType.DMA,
        ],
    )


@jax.jit
def kernel(x, tables):
    xt = x.T
    tbl = tables.reshape(N_LEVELS * T * F)
    out = _make_sc_call()(xt, tbl)
    return out.T


# speedup vs baseline: 100.7174x; 5.1614x over previous
import functools

import numpy as np
import jax
import jax.numpy as jnp
from jax import lax
from jax.experimental import pallas as pl
from jax.experimental.pallas import tpu as pltpu
from jax.experimental.pallas import tpu_sc as plsc

N_LEVELS = 16
F = 2
LOG2_T = 19
T = 2 ** LOG2_T
HASH_MASK = T - 1
BASE_RES = 16.0
FINEST_RES = 512.0
N_PTS = 262144
B_GROWTH = float(np.exp((np.log(FINEST_RES) - np.log(BASE_RES)) / (N_LEVELS - 1)))
_RES = [float(np.floor(np.float32(BASE_RES * (B_GROWTH ** i)))) for i in range(N_LEVELS)]
_GS = [float(np.float32(2.0) / np.float32(r)) for r in _RES]
P1_I = 2654435761 - 2 ** 32
P2_I = 805459861

NC = 2
NS = 16
NW = NC * NS
PW = N_PTS // NW
C = 512
NCHUNK = PW // C
GROUPS = C // 16
NIDX = F * 8 * C


def _sc_body(xt_hbm, tbl_hbm, out_hbm, x_v, w0_v, w1_v, idx0_v, idx1_v,
             rows0_v, rows1_v, out_v, sem0, sem1):
    c_id = lax.axis_index("c")
    s_id = lax.axis_index("s")
    wid = s_id * NC + c_id

    def make_phase_a(lvl, idx_ref, w_ref):
        gs = jnp.float32(_GS[lvl])
        lvl_off2 = jnp.int32(2 * lvl * T)

        def phase_a(g, carry):
            p0 = pl.multiple_of(g * 16, 16)
            bl = []
            for d in range(3):
                xd = x_v[d, pl.ds(p0, 16)]
                s = (xd - jnp.float32(-1.0)) / gs
                bli = s.astype(jnp.int32)
                blf = bli.astype(jnp.float32)
                vmin = blf * gs + jnp.float32(-1.0)
                den = (vmin + gs) - vmin
                w_ref[d, pl.ds(p0, 16)] = (xd - vmin) / den
                bl.append(bli)
            m0 = bl[0]
            m0b = m0 + jnp.int32(1)
            m1 = bl[1] * jnp.int32(P1_I)
            m1b = m1 + jnp.int32(P1_I)
            m2 = bl[2] * jnp.int32(P2_I)
            m2b = m2 + jnp.int32(P2_I)
            e00 = m0 ^ m1
            e01 = m0 ^ m1b
            e10 = m0b ^ m1
            e11 = m0b ^ m1b
            corners = (e00 ^ m2, e00 ^ m2b, e01 ^ m2, e01 ^ m2b,
                       e10 ^ m2, e10 ^ m2b, e11 ^ m2, e11 ^ m2b)
            for c, h in enumerate(corners):
                t = h & jnp.int32(HASH_MASK)
                f0 = ((t >> 7) << 8) + (t & jnp.int32(127)) + lvl_off2
                idx_ref[pl.ds(c * C + p0, 16)] = f0
                idx_ref[pl.ds(8 * C + c * C + p0, 16)] = f0 + jnp.int32(128)
            return carry

        lax.fori_loop(0, GROUPS, phase_a, 0)

    def make_phase_b(lvl, rows_ref, w_ref):
        def phase_b(g, carry):
            p0 = pl.multiple_of(g * 16, 16)
            wx = w_ref[0, pl.ds(p0, 16)]
            wy = w_ref[1, pl.ds(p0, 16)]
            wz = w_ref[2, pl.ds(p0, 16)]
            for f in range(F):
                e = [rows_ref[pl.ds(f * 8 * C + c * C + p0, 16)] for c in range(8)]
                c00 = e[0] + wx * (e[4] - e[0])
                c01 = e[1] + wx * (e[5] - e[1])
                c10 = e[2] + wx * (e[6] - e[2])
                c11 = e[3] + wx * (e[7] - e[3])
                c0 = c00 + wy * (c10 - c00)
                c1 = c01 + wy * (c11 - c01)
                out_v[2 * lvl + f, pl.ds(p0, 16)] = c0 + wz * (c1 - c0)
            return carry

        lax.fori_loop(0, GROUPS, phase_b, 0)

    bufs = [(idx0_v, rows0_v, w0_v, sem0), (idx1_v, rows1_v, w1_v, sem1)]

    def chunk_body(ch, carry):
        base = (wid * NCHUNK + ch) * C
        pltpu.sync_copy(xt_hbm.at[:, pl.ds(base, C)], x_v)

        idx_r, rows_r, w_r, sem_r = bufs[0]
        make_phase_a(0, idx_r, w_r)
        cps = [pltpu.async_copy(tbl_hbm.at[idx_r], rows_r, sem_r)]
        for lvl in range(N_LEVELS):
            if lvl + 1 < N_LEVELS:
                idx_n, rows_n, w_n, sem_n = bufs[(lvl + 1) % 2]
                make_phase_a(lvl + 1, idx_n, w_n)
                cps.append(pltpu.async_copy(tbl_hbm.at[idx_n], rows_n, sem_n))
            idx_l, rows_l, w_l, sem_l = bufs[lvl % 2]
            cps[lvl].wait()
            make_phase_b(lvl, rows_l, w_l)
        pltpu.sync_copy(out_v, out_hbm.at[:, pl.ds(base, C)])
        return carry

    lax.fori_loop(0, NCHUNK, chunk_body, 0)


@functools.lru_cache(maxsize=1)
def _make_sc_call():
    mesh = plsc.VectorSubcoreMesh(
        core_axis_name="c", subcore_axis_name="s", num_cores=NC, num_subcores=NS
    )
    return pl.kernel(
        _sc_body,
        out_type=jax.ShapeDtypeStruct((2 * N_LEVELS, N_PTS), jnp.float32),
        mesh=mesh,
        scratch_types=[
            pltpu.VMEM((3, C), jnp.float32),
            pltpu.VMEM((3, C), jnp.float32),
            pltpu.VMEM((3, C), jnp.float32),
            pltpu.VMEM((NIDX,), jnp.int32),
            pltpu.VMEM((NIDX,), jnp.int32),
            pltpu.VMEM((NIDX,), jnp.float32),
            pltpu.VMEM((NIDX,), jnp.float32),
            pltpu.VMEM((2 * N_LEVELS, C), jnp.float32),
            pltpu.SemaphoreType.DMA,
            pltpu.SemaphoreType.DMA,
        ],
    )


@jax.jit
def kernel(x, tables):
    xt = x.T
    tbl = tables.reshape(N_LEVELS, T // 128, 128, F).transpose(0, 1, 3, 2).reshape(-1)
    out = _make_sc_call()(xt, tbl)
    return out.T
